# Initial kernel scaffold; baseline (speedup 1.0000x reference)
import jax
import jax.numpy as jnp
from jax.experimental import pallas as pl


def kernel(x, edge_index, W0, b0, W1, b1, W2, b2, W3, b3, Wfc, bfc):
    raise NotImplementedError("write your pallas kernel here")


# speedup vs baseline: 5.1202x; 5.1202x over previous
import functools

import jax
import jax.numpy as jnp
from jax import lax
from jax.experimental import pallas as pl
from jax.experimental.pallas import tpu as pltpu
from jax.experimental.pallas import tpu_sc as plsc

_N = 10000
_E = 160000
_D = 256
_C = 40
_HALF = 128
_NC = 2
_NS = 16
_NPAD = 10240
_K = 125
_NCH = _E // _NS // _K
_RPT = _NPAD // _NS
_RB = 1000


def _segsum_sc(h_flat, src_idx, dst_idx, zeros):
    mesh = plsc.VectorSubcoreMesh(core_axis_name="c", subcore_axis_name="s")

    @functools.partial(
        pl.kernel,
        mesh=mesh,
        out_type=jax.ShapeDtypeStruct((_NC * _NPAD, _HALF), jnp.float32),
        scratch_types=[
            pltpu.VMEM((_NCH, _K), jnp.int32),
            pltpu.VMEM((_NCH, _K), jnp.int32),
            pltpu.VMEM((_K, _HALF), jnp.float32),
            pltpu.VMEM_SHARED((_NPAD, _HALF), jnp.float32),
        ],
    )
    def seg(h_hbm, src_hbm, dst_hbm, z_hbm, out_hbm, srcv, dstv, buf, acc):
        c = lax.axis_index("c")
        s = lax.axis_index("s")
        pltpu.sync_copy(z_hbm.at[pl.ds(s * _RPT, _RPT)],
                        acc.at[pl.ds(s * _RPT, _RPT)])
        pltpu.sync_copy(src_hbm.at[c, s], srcv)
        pltpu.sync_copy(dst_hbm.at[s], dstv)
        plsc.subcore_barrier()

        def chunk(j, carry):
            pltpu.sync_copy(h_hbm.at[srcv.at[j]], buf)
            pltpu.sync_copy(buf, acc.at[dstv.at[j]], add=True)
            return carry

        lax.fori_loop(0, _NCH, chunk, 0)
        plsc.subcore_barrier()
        pltpu.sync_copy(acc.at[pl.ds(s * _RPT, _RPT)],
                        out_hbm.at[pl.ds(c * _NPAD + s * _RPT, _RPT)])

    return seg(h_flat, src_idx, dst_idx, zeros)


def _tc_layer(h_split, agg_split, w_split, b_row):
    def body(h_ref, a_ref, w_ref, b_ref, o_ref):
        x0 = h_ref[0] + a_ref[0]
        x1 = h_ref[1] + a_ref[1]
        z = jnp.dot(x0, w_ref[0], preferred_element_type=jnp.float32)
        z = z + jnp.dot(x1, w_ref[1], preferred_element_type=jnp.float32)
        z = jnp.maximum(z + b_ref[0], 0.0)
        o_ref[0] = z[:, :_HALF]
        o_ref[1] = z[:, _HALF:]

    return pl.pallas_call(
        body,
        grid=(_N // _RB,),
        in_specs=[
            pl.BlockSpec((_NC, _RB, _HALF), lambda i: (0, i, 0)),
            pl.BlockSpec((_NC, _RB, _HALF), lambda i: (0, i, 0)),
            pl.BlockSpec((_NC, _HALF, _D), lambda i: (0, 0, 0)),
            pl.BlockSpec((1, _D), lambda i: (0, 0)),
        ],
        out_specs=pl.BlockSpec((_NC, _RB, _HALF), lambda i: (0, i, 0)),
        out_shape=jax.ShapeDtypeStruct((_NC, _NPAD, _HALF), jnp.float32),
    )(h_split, agg_split, w_split, b_row)


def _tc_final(h_split, agg_split, w_split, b_row, wfc, bfc_row):
    def body(h_ref, a_ref, w_ref, b_ref, wfc_ref, bfc_ref, o_ref):
        x0 = h_ref[0] + a_ref[0]
        x1 = h_ref[1] + a_ref[1]
        z = jnp.dot(x0, w_ref[0], preferred_element_type=jnp.float32)
        z = z + jnp.dot(x1, w_ref[1], preferred_element_type=jnp.float32)
        z = jnp.maximum(z + b_ref[0], 0.0)
        o_ref[...] = (jnp.dot(z, wfc_ref[...], preferred_element_type=jnp.float32)
                      + bfc_ref[0])

    return pl.pallas_call(
        body,
        grid=(_N // _RB,),
        in_specs=[
            pl.BlockSpec((_NC, _RB, _HALF), lambda i: (0, i, 0)),
            pl.BlockSpec((_NC, _RB, _HALF), lambda i: (0, i, 0)),
            pl.BlockSpec((_NC, _HALF, _D), lambda i: (0, 0, 0)),
            pl.BlockSpec((1, _D), lambda i: (0, 0)),
            pl.BlockSpec((_D, _C), lambda i: (0, 0)),
            pl.BlockSpec((1, _C), lambda i: (0, 0)),
        ],
        out_specs=pl.BlockSpec((_RB, _C), lambda i: (i, 0)),
        out_shape=jax.ShapeDtypeStruct((_N, _C), jnp.float32),
    )(h_split, agg_split, w_split, b_row, wfc, bfc_row)


def kernel(x, edge_index, W0, b0, W1, b1, W2, b2, W3, b3, Wfc, bfc):
    src = edge_index[0]
    dst = edge_index[1]
    srcg = jnp.reshape(jnp.stack([src, src + _NPAD]), (_NC, _NS, _NCH, _K))
    dstg = jnp.reshape(dst, (_NS, _NCH, _K))
    zeros = jnp.zeros((_NPAD, _HALF), jnp.float32)

    h = jnp.reshape(
        jnp.pad(jnp.transpose(jnp.reshape(x, (_N, _NC, _HALF)), (1, 0, 2)),
                ((0, 0), (0, _NPAD - _N), (0, 0))),
        (_NC * _NPAD, _HALF))

    for W, b in ((W0, b0), (W1, b1), (W2, b2)):
        agg = _segsum_sc(h, srcg, dstg, zeros)
        h = jnp.reshape(
            _tc_layer(jnp.reshape(h, (_NC, _NPAD, _HALF)),
                      jnp.reshape(agg, (_NC, _NPAD, _HALF)),
                      jnp.reshape(W, (_NC, _HALF, _D)),
                      jnp.reshape(b, (1, _D))),
            (_NC * _NPAD, _HALF))

    agg = _segsum_sc(h, srcg, dstg, zeros)
    return _tc_final(jnp.reshape(h, (_NC, _NPAD, _HALF)),
                     jnp.reshape(agg, (_NC, _NPAD, _HALF)),
                     jnp.reshape(W3, (_NC, _HALF, _D)),
                     jnp.reshape(b3, (1, _D)),
                     Wfc,
                     jnp.reshape(bfc, (1, _C)))
